# baseline (speedup 1.0000x reference)
import functools

import jax
import jax.numpy as jnp
from jax import lax
from jax.experimental import pallas as pl
from jax.experimental.pallas import tpu as pltpu
from jax.experimental.pallas import tpu_sc as plsc

ROWS = 16384
SEQ = 50
DIM = 64
NC, NS = 2, 16
NW = NC * NS
TCG = ROWS // 128
GPW = TCG // NW
L = 16


def _emb_body(idx_hbm, table_hbm, out_hbm, idx_v, idx_t, rows, chunk, sg, sw):
    wid = lax.axis_index("s") * NC + lax.axis_index("c")
    iota = lax.iota(jnp.int32, L)

    def transpose_idx(_):
        def per_q(q, _):
            qvec = jnp.full((L,), q, jnp.int32)
            for lg in range(8):
                v = plsc.load_gather(idx_v, [lg * L + iota, qvec])
                idx_t[q, pl.ds(lg * L, L)] = v
            return _
        lax.fori_loop(0, SEQ, per_q, None)

    def gather(q, b):
        pltpu.async_copy(table_hbm.at[idx_t.at[q]], rows[b], sg[b])

    def gather_wait(b):
        pltpu.make_async_copy(table_hbm.at[idx_t.at[0]], rows[b], sg[b]).wait()

    row_idx = [lg * L + iota for lg in range(8)]

    def transpose_block(b):
        for tr in range(8):
            for s in range(8):
                dvec = jnp.full((L,), 8 * tr + s, jnp.int32)
                for lg in range(8):
                    v = plsc.load_gather(rows[b], [row_idx[lg], dvec])
                    chunk[b][tr, s, pl.ds(lg * L, L)] = v

    def write(q, tc, b):
        for tr in range(8):
            pltpu.async_copy(chunk[b].at[tr], out_hbm.at[q, tr, tc], sw[b])

    def write_wait(b):
        for tr in range(8):
            pltpu.make_async_copy(chunk[b].at[tr], out_hbm.at[0, tr, 0],
                                  sw[b]).wait()

    def per_group(j, _):
        tc = wid * GPW + j
        pltpu.sync_copy(idx_hbm.at[pl.ds(tc * 128, 128)], idx_v)
        transpose_idx(None)
        gather(0, 0)

        def pair(g, _):
            for b in range(2):
                q = 2 * g + b

                @pl.when(q < SEQ - 1)
                def _():
                    gather(q + 1, 1 - b)

                gather_wait(b)

                @pl.when(q >= 2)
                def _():
                    write_wait(b)

                transpose_block(b)
                write(q, tc, b)
            return _

        lax.fori_loop(0, SEQ // 2, pair, None)
        write_wait(0)
        write_wait(1)
        return _

    lax.fori_loop(0, GPW, per_group, None)


@jax.jit
def _embedding_lookup(idx, weight):
    mesh = plsc.VectorSubcoreMesh(core_axis_name="c", subcore_axis_name="s")
    k = functools.partial(
        pl.kernel,
        out_type=jax.ShapeDtypeStruct((SEQ, 8, TCG, 8, 128), jnp.float32),
        mesh=mesh,
        scratch_types=[
            pltpu.VMEM((128, SEQ), jnp.int32),
            pltpu.VMEM((SEQ, 128), jnp.int32),
            [pltpu.VMEM((128, DIM), jnp.float32) for _ in range(2)],
            [pltpu.VMEM((8, 8, 128), jnp.float32) for _ in range(2)],
            [pltpu.SemaphoreType.DMA for _ in range(2)],
            [pltpu.SemaphoreType.DMA for _ in range(2)],
        ],
        compiler_params=pltpu.CompilerParams(use_tc_tiling_on_sc=False,
                                             needs_layout_passes=False),
    )(_emb_body)
    out5 = k(idx, weight)
    return out5.transpose(2, 4, 0, 1, 3).reshape(ROWS, SEQ, DIM)


def kernel(token_ids, weight):
    return _embedding_lookup(token_ids.astype(jnp.int32), weight)


# speedup vs baseline: 1.6206x; 1.6206x over previous
import functools

import jax
import jax.numpy as jnp
from jax import lax
from jax.experimental import pallas as pl
from jax.experimental.pallas import tpu as pltpu
from jax.experimental.pallas import tpu_sc as plsc

ROWS = 16384
SEQ = 50
DIM = 64
NC, NS = 2, 16
NW = NC * NS
RPW = ROWS // NW
NBUF = 8
LAG = NBUF // 2


def _emb_body(idx_hbm, table_hbm, out_hbm, idx_v, rows, sg, so):
    wid = lax.axis_index("s") * NC + lax.axis_index("c")
    base = wid * RPW
    pltpu.sync_copy(idx_hbm.at[pl.ds(base, RPW)], idx_v)

    def gather(t, b):
        pltpu.async_copy(table_hbm.at[idx_v.at[t]], rows[b], sg[b])

    def gather_wait(t, b):
        pltpu.make_async_copy(table_hbm.at[idx_v.at[t]], rows[b], sg[b]).wait()

    def write_out(t, b):
        pltpu.async_copy(rows[b], out_hbm.at[base + t], so[b])

    def write_wait(t, b):
        pltpu.make_async_copy(rows[b], out_hbm.at[base + t], so[b]).wait()

    for i in range(NBUF):
        gather(i, i)
        if i >= LAG:
            j = i - LAG
            gather_wait(j, j)
            write_out(j, j)

    def group(g, _):
        for b in range(NBUF):
            i = NBUF * g + b
            j = i - LAG
            bj = (b - LAG) % NBUF
            write_wait(i - NBUF, b)
            gather(i, b)
            gather_wait(j, bj)
            write_out(j, bj)
        return _

    lax.fori_loop(1, RPW // NBUF, group, None)

    for j in range(RPW - LAG, RPW):
        bj = j % NBUF
        gather_wait(j, bj)
        write_out(j, bj)
    for j in range(RPW - NBUF, RPW):
        write_wait(j, j % NBUF)


@jax.jit
def _embedding_lookup(idx, weight):
    mesh = plsc.VectorSubcoreMesh(core_axis_name="c", subcore_axis_name="s")
    k = functools.partial(
        pl.kernel,
        out_type=jax.ShapeDtypeStruct((ROWS, SEQ, DIM), jnp.float32),
        mesh=mesh,
        scratch_types=[
            pltpu.VMEM((RPW, SEQ), jnp.int32),
            [pltpu.VMEM((SEQ, DIM), jnp.float32) for _ in range(NBUF)],
            [pltpu.SemaphoreType.DMA for _ in range(NBUF)],
            [pltpu.SemaphoreType.DMA for _ in range(NBUF)],
        ],
        compiler_params=pltpu.CompilerParams(use_tc_tiling_on_sc=False),
    )(_emb_body)
    return k(idx, weight)


def kernel(token_ids, weight):
    return _embedding_lookup(token_ids.astype(jnp.int32), weight)
